# speedup vs baseline: 2.8998x; 2.8998x over previous
import functools

import jax
import jax.numpy as jnp
from jax import lax
from jax.experimental import pallas as pl
from jax.experimental.pallas import tpu as pltpu
from jax.experimental.pallas import tpu_sc as plsc

D_MODEL = 128
NC, NS = 2, 16
NW = NC * NS
CHUNK = 128


def _emb_body(idx_hbm, table_hbm, out_hbm, idx_v, rows_v, gsem):
    wid = lax.axis_index("s") * NC + lax.axis_index("c")
    n_chunks = idx_hbm.shape[1]
    pltpu.sync_copy(idx_hbm.at[wid], idx_v)

    def step(j, carry):
        pltpu.async_copy(table_hbm.at[idx_v.at[j]], rows_v, gsem).wait()
        pltpu.sync_copy(rows_v, out_hbm.at[wid, j])
        return carry

    lax.fori_loop(0, n_chunks, step, 0, unroll=False)


@jax.jit
def kernel(x, table):
    B, F = x.shape
    n_total = B * F
    assert n_total % (NW * CHUNK) == 0
    n_chunks = n_total // (NW * CHUNK)
    idx = x.astype(jnp.int32).reshape(NW, n_chunks, CHUNK)

    mesh = plsc.VectorSubcoreMesh(core_axis_name="c", subcore_axis_name="s")
    run = pl.kernel(
        _emb_body,
        out_type=jax.ShapeDtypeStruct((NW, n_chunks, CHUNK, D_MODEL), jnp.float32),
        mesh=mesh,
        scratch_types=[
            pltpu.VMEM((n_chunks, CHUNK), jnp.int32),
            pltpu.VMEM((CHUNK, D_MODEL), jnp.float32),
            pltpu.SemaphoreType.DMA,
        ],
    )
    out = run(idx, table)
    return out.reshape(B, F, D_MODEL)


# speedup vs baseline: 4.4325x; 1.5286x over previous
import functools

import jax
import jax.numpy as jnp
from jax import lax
from jax.experimental import pallas as pl
from jax.experimental.pallas import tpu as pltpu
from jax.experimental.pallas import tpu_sc as plsc

D_MODEL = 128
NC, NS = 2, 16
NW = NC * NS
CHUNK = 100
NBUF = 4


def _emb_body(idx_hbm, table_hbm, out_hbm, idx_v, rows, gsem, ssem):
    wid = lax.axis_index("s") * NC + lax.axis_index("c")
    n_chunks = idx_hbm.shape[1]
    pltpu.sync_copy(idx_hbm.at[wid], idx_v)

    def gather(j, b):
        return pltpu.make_async_copy(
            table_hbm.at[idx_v.at[j]], rows.at[b], gsem.at[b])

    def store(j, b):
        return pltpu.make_async_copy(
            rows.at[b], out_hbm.at[wid, j], ssem.at[b])

    for b in range(NBUF):
        gather(b, b).start()

    def group(g, carry):
        for b in range(NBUF):
            j = g * NBUF + b
            gather(j, b).wait()
            store(j, b).start()
            jn = j + NBUF - 1
            bn = (b - 1) % NBUF

            @pl.when(jnp.logical_and(j >= 1, jn < n_chunks))
            def _():
                store(j - 1, bn).wait()
                gather(jn, bn).start()

        return carry

    lax.fori_loop(0, n_chunks // NBUF, group, 0, unroll=False)
    for b in range(NBUF):
        store(n_chunks - NBUF + b, b).wait()


@jax.jit
def kernel(x, table):
    B, F = x.shape
    n_total = B * F
    assert n_total % (NW * CHUNK * NBUF) == 0
    n_chunks = n_total // (NW * CHUNK)
    idx = x.astype(jnp.int32).reshape(NW, n_chunks, CHUNK)

    mesh = plsc.VectorSubcoreMesh(core_axis_name="c", subcore_axis_name="s")
    run = pl.kernel(
        _emb_body,
        out_type=jax.ShapeDtypeStruct((NW, n_chunks, CHUNK, D_MODEL), jnp.float32),
        mesh=mesh,
        scratch_types=[
            pltpu.VMEM((n_chunks, CHUNK), jnp.int32),
            pltpu.VMEM((NBUF, CHUNK, D_MODEL), jnp.float32),
            pltpu.SemaphoreType.DMA((NBUF,)),
            pltpu.SemaphoreType.DMA((NBUF,)),
        ],
    )
    out = run(idx, table)
    return out.reshape(B, F, D_MODEL)
